# baseline (speedup 1.0000x reference)
import functools
import jax
import jax.numpy as jnp
from jax import lax
from jax.experimental import pallas as pl
from jax.experimental.pallas import tpu as pltpu
from jax.experimental.pallas import tpu_sc as plsc

B = 16384
D = 64
L = 16
CH = 256


@jax.jit
def _bprmf_gather(user, pos_item, neg_item, embed_user, embed_item):
    ei2 = embed_item.reshape(embed_item.shape[0] // 2, 2 * D)

    info = plsc.get_sparse_core_info()
    nc, ns = info.num_cores, info.num_subcores
    nw = nc * ns
    bpw = B // nw
    mesh = plsc.VectorSubcoreMesh(core_axis_name="c", subcore_axis_name="s")

    @functools.partial(
        pl.kernel,
        mesh=mesh,
        out_type=jax.ShapeDtypeStruct((B, D), jnp.float32),
        scratch_types=[
            pltpu.VMEM((bpw,), jnp.int32),
            pltpu.VMEM((bpw, D), jnp.float32),
            pltpu.SemaphoreType.DMA,
        ],
    )
    def gather_user(idx_hbm, tab_hbm, out_hbm, idx_v, rows_v, sem):
        wid = lax.axis_index("s") * nc + lax.axis_index("c")
        base = wid * bpw
        pltpu.sync_copy(idx_hbm.at[pl.ds(base, bpw)], idx_v)

        @plsc.parallel_loop(0, bpw // L, unroll=2)
        def group_body(g):
            v16 = idx_v[pl.ds(g * L, L)]
            for jj in range(L):
                pltpu.async_copy(
                    tab_hbm.at[v16[jj]], rows_v.at[g * L + jj], sem)

        pltpu.make_async_copy(tab_hbm.at[pl.ds(0, bpw)], rows_v, sem).wait()
        pltpu.sync_copy(rows_v, out_hbm.at[pl.ds(base, bpw)])

    @functools.partial(
        pl.kernel,
        mesh=mesh,
        out_type=(
            jax.ShapeDtypeStruct((B, D), jnp.float32),
            jax.ShapeDtypeStruct((B, D), jnp.float32),
        ),
        scratch_types=[
            pltpu.VMEM((bpw,), jnp.int32),
            pltpu.VMEM((bpw,), jnp.int32),
            pltpu.VMEM((CH, 2 * D), jnp.float32),
            pltpu.VMEM((CH, D), jnp.float32),
            pltpu.SemaphoreType.DMA,
        ],
    )
    def gather_items(pos_hbm, neg_hbm, tab_hbm,
                     out_p, out_n, idx_v, fix_v, buf, outb, sem):
        wid = lax.axis_index("s") * nc + lax.axis_index("c")
        base = wid * bpw

        def one_batch(idx_hbm, out_hbm):
            pltpu.sync_copy(idx_hbm.at[pl.ds(base, bpw)], idx_v)

            @plsc.parallel_loop(0, bpw // L, unroll=4)
            def fix_body(m):
                fix_v[pl.ds(m * L, L)] = lax.shift_right_logical(
                    idx_v[pl.ds(m * L, L)], 1)

            def chunk_body(c, _):
                cp = pltpu.async_copy(
                    tab_hbm.at[fix_v.at[pl.ds(c * CH, CH)]], buf, sem)
                cp.wait()

                def sel_body(g, _):
                    j0 = g * L
                    off16 = (idx_v[pl.ds(c * CH + j0, L)] & 1) * D
                    for jj in range(L):
                        o = off16[jj]
                        for kk in range(D // L):
                            outb[j0 + jj, pl.ds(kk * L, L)] = (
                                buf[j0 + jj, pl.ds(o + kk * L, L)])
                    return _
                lax.fori_loop(0, CH // L, sel_body, 0)
                pltpu.sync_copy(outb, out_hbm.at[pl.ds(base + c * CH, CH)])
                return _
            lax.fori_loop(0, bpw // CH, chunk_body, 0)

        one_batch(pos_hbm, out_p)
        one_batch(neg_hbm, out_n)

    out_u = gather_user(user, embed_user)
    out_p, out_n = gather_items(pos_item, neg_item, ei2)
    return out_u, out_p, out_n


def kernel(user, pos_item, neg_item, embed_user, embed_item):
    return _bprmf_gather(user, pos_item, neg_item, embed_user, embed_item)


# speedup vs baseline: 1.2582x; 1.2582x over previous
import functools
import jax
import jax.numpy as jnp
from jax import lax
from jax.experimental import pallas as pl
from jax.experimental.pallas import tpu as pltpu
from jax.experimental.pallas import tpu_sc as plsc

B = 16384
D = 64
L = 16


@jax.jit
def _bprmf_gather(user, pos_item, neg_item, embed_user, embed_item):
    info = plsc.get_sparse_core_info()
    nc, ns = info.num_cores, info.num_subcores
    nw = nc * ns
    bpw = B // nw
    mesh = plsc.VectorSubcoreMesh(core_axis_name="c", subcore_axis_name="s")

    @functools.partial(
        pl.kernel,
        mesh=mesh,
        out_type=(
            jax.ShapeDtypeStruct((B, D), jnp.float32),
            jax.ShapeDtypeStruct((B, D), jnp.float32),
            jax.ShapeDtypeStruct((B, D), jnp.float32),
        ),
        scratch_types=[
            pltpu.VMEM((bpw,), jnp.int32),
            pltpu.VMEM((bpw, D), jnp.float32),
            pltpu.SemaphoreType.DMA,
        ],
    )
    def k(user_hbm, pos_hbm, neg_hbm, eu_hbm, ei_hbm,
          out_u, out_p, out_n, idx_v, rows_v, sem):
        wid = lax.axis_index("s") * nc + lax.axis_index("c")
        base = wid * bpw

        def one_table(idx_hbm, tab_hbm, out_hbm):
            pltpu.sync_copy(idx_hbm.at[pl.ds(base, bpw)], idx_v)

            @plsc.parallel_loop(0, bpw // L, unroll=2)
            def group_body(g):
                v16 = idx_v[pl.ds(g * L, L)]
                for jj in range(L):
                    pltpu.async_copy(
                        tab_hbm.at[v16[jj]], rows_v.at[g * L + jj], sem)

            pltpu.make_async_copy(
                tab_hbm.at[pl.ds(0, bpw)], rows_v, sem).wait()
            pltpu.sync_copy(rows_v, out_hbm.at[pl.ds(base, bpw)])

        one_table(user_hbm, eu_hbm, out_u)
        one_table(pos_hbm, ei_hbm, out_p)
        one_table(neg_hbm, ei_hbm, out_n)

    return k(user, pos_item, neg_item, embed_user, embed_item)


def kernel(user, pos_item, neg_item, embed_user, embed_item):
    return _bprmf_gather(user, pos_item, neg_item, embed_user, embed_item)
